# Initial kernel scaffold; baseline (speedup 1.0000x reference)
import jax
import jax.numpy as jnp
from jax.experimental import pallas as pl


def kernel(x, x_shuf, edge_index, W_o1, b_o1, W_o2, b_o2, W_s1, b_s1, W_s2, b_s2, W_disc, b_disc):
    raise NotImplementedError("write your pallas kernel here")


# speedup vs baseline: 6.1432x; 6.1432x over previous
import functools

import jax
import jax.numpy as jnp
from jax import lax
from jax.experimental import pallas as pl
from jax.experimental.pallas import tpu as pltpu
from jax.experimental.pallas import tpu_sc as plsc

D = 128
R = 256
ZR = 64
CH = 128
DEG_CH = 2048


def _mesh():
    return plsc.VectorSubcoreMesh(core_axis_name="c", subcore_axis_name="s")


def _make_deg_kernel(EP, NP):
    per_w = EP // 32

    @functools.partial(
        pl.kernel,
        mesh=_mesh(),
        out_type=jax.ShapeDtypeStruct((32, NP), jnp.float32),
        scratch_types=[
            pltpu.VMEM((NP,), jnp.float32),
            pltpu.VMEM((DEG_CH,), jnp.int32),
        ],
        compiler_params=pltpu.CompilerParams(needs_layout_passes=False),
    )
    def deg_kernel(dst_hbm, out_hbm, acc_v, ids_v):
        cid = lax.axis_index("c")
        sid = lax.axis_index("s")
        wid = sid * 2 + cid
        base = wid * per_w

        def zero_body(i, carry):
            acc_v[pl.ds(i * 16, 16)] = jnp.zeros((16,), jnp.float32)
            return carry

        lax.fori_loop(0, NP // 16, zero_body, 0)

        ones = jnp.ones((16,), jnp.float32)

        def chunk_body(c, carry):
            pltpu.sync_copy(dst_hbm.at[pl.ds(base + c * DEG_CH, DEG_CH)], ids_v)

            def scat_body(i, carry2):
                idx = ids_v[pl.ds(i * 16, 16)]
                plsc.addupdate_scatter(acc_v, [idx], ones)
                return carry2

            lax.fori_loop(0, DEG_CH // 16, scat_body, 0)
            return carry

        lax.fori_loop(0, per_w // DEG_CH, chunk_body, 0)
        pltpu.sync_copy(acc_v, out_hbm.at[wid])

    return deg_kernel


def _make_prop_kernel(T, EP, NP):
    per_w = EP // 32
    n_chunks = per_w // CH
    rows_w = NP // 16

    out_types = [jax.ShapeDtypeStruct((2, NP, D), jnp.float32) for _ in range(T)]
    scratch_types = [
        pltpu.VMEM((CH,), jnp.int32),
        pltpu.VMEM((CH,), jnp.int32),
        pltpu.VMEM((CH, D), jnp.float32),
        pltpu.VMEM((ZR, D), jnp.float32),
        pltpu.VMEM_SHARED((NP, D), jnp.float32),
        pltpu.SemaphoreType.DMA,
    ]

    @functools.partial(
        pl.kernel,
        mesh=_mesh(),
        out_type=out_types,
        scratch_types=scratch_types,
    )
    def prop_kernel(src_hbm, dst_hbm, zeros_hbm, *rest):
        tables = rest[:T]
        outs = rest[T:2 * T]
        sidx_v, didx_v, rows_v, zero_v, acc_sh, sem = rest[2 * T:]

        cid = lax.axis_index("c")
        sid = lax.axis_index("s")
        wid = sid * 2 + cid
        ebase = wid * per_w
        row0 = sid * rows_w

        pltpu.sync_copy(zeros_hbm, zero_v)

        for t in range(T):
            def zero_body(j, carry):
                pltpu.sync_copy(zero_v, acc_sh.at[pl.ds(row0 + j * ZR, ZR)])
                return carry

            lax.fori_loop(0, rows_w // ZR, zero_body, 0)
            plsc.subcore_barrier()

            table = tables[t]

            def edge_body(c, carry):
                off = ebase + c * CH
                pltpu.sync_copy(src_hbm.at[pl.ds(off, CH)], sidx_v)
                pltpu.sync_copy(dst_hbm.at[pl.ds(off, CH)], didx_v)
                pltpu.async_copy(table.at[sidx_v], rows_v, sem).wait()
                pltpu.sync_copy(rows_v, acc_sh.at[didx_v], add=True)
                return carry

            lax.fori_loop(0, n_chunks, edge_body, 0)
            plsc.subcore_barrier()

            def flush_body(j, carry):
                r = row0 + j * CH
                pltpu.sync_copy(acc_sh.at[pl.ds(r, CH)], rows_v)
                pltpu.sync_copy(rows_v, outs[t].at[cid, pl.ds(r, CH)])
                return carry

            lax.fori_loop(0, rows_w // CH, flush_body, 0)

    return prop_kernel


def _dinv_from_partials(p_blk):
    deg = jnp.sum(p_blk, axis=0) + 1.0
    return lax.rsqrt(deg)[:, None]


def _tc_scale(partials, xp, xsp, NP):
    def body(p_ref, x_ref, xs_ref, gx_ref, gs_ref):
        dinv = _dinv_from_partials(p_ref[...])
        gx_ref[...] = x_ref[...] * dinv
        gs_ref[...] = xs_ref[...] * dinv

    grid = NP // R
    blk = lambda r: (r, 0)
    return pl.pallas_call(
        body,
        grid=(grid,),
        in_specs=[
            pl.BlockSpec((32, R), lambda r: (0, r)),
            pl.BlockSpec((R, D), blk),
            pl.BlockSpec((R, D), blk),
        ],
        out_specs=[pl.BlockSpec((R, D), blk)] * 2,
        out_shape=[jax.ShapeDtypeStruct((NP, D), jnp.float32)] * 2,
    )(partials, xp, xsp)


def _tc_layer1(partials, acc1x, acc1s, g1x, g1s, W_o1, W_s1, b_o1, b_s1, NP):
    def body(p_ref, ax_ref, as_ref, gx_ref, gs_ref,
             wo_ref, ws_ref, bo_ref, bs_ref,
             o1_ref, o2_ref, o3_ref, o4_ref):
        dinv = _dinv_from_partials(p_ref[...])
        p1x = (ax_ref[0] + ax_ref[1] + gx_ref[...]) * dinv
        p1s = (as_ref[0] + as_ref[1] + gs_ref[...]) * dinv
        wo = wo_ref[...]
        ws = ws_ref[...]
        bo = bo_ref[...]
        bs = bs_ref[...]
        dot = lambda a, w: jnp.dot(a, w, preferred_element_type=jnp.float32)
        o1_ref[...] = jax.nn.relu(dot(p1x, wo) + bo) * dinv
        o2_ref[...] = jax.nn.relu(dot(p1x, ws) + bs) * dinv
        o3_ref[...] = jax.nn.relu(dot(p1s, wo) + bo) * dinv
        o4_ref[...] = jax.nn.relu(dot(p1s, ws) + bs) * dinv

    grid = NP // R
    blk = lambda r: (r, 0)
    acc_spec = pl.BlockSpec((2, R, D), lambda r: (0, r, 0))
    w_spec = pl.BlockSpec((D, D), lambda r: (0, 0))
    b_spec = pl.BlockSpec((1, D), lambda r: (0, 0))
    return pl.pallas_call(
        body,
        grid=(grid,),
        in_specs=[
            pl.BlockSpec((32, R), lambda r: (0, r)),
            acc_spec, acc_spec,
            pl.BlockSpec((R, D), blk), pl.BlockSpec((R, D), blk),
            w_spec, w_spec, b_spec, b_spec,
        ],
        out_specs=[pl.BlockSpec((R, D), blk)] * 4,
        out_shape=[jax.ShapeDtypeStruct((NP, D), jnp.float32)] * 4,
    )(partials, acc1x, acc1s, g1x, g1s, W_o1, W_s1, b_o1, b_s1)


def _tc_layer2(partials, accs, gs, W_o2, W_s2, b_o2, b_s2, NP, N):
    def body(p_ref, a1, a2, a3, a4, g1, g2, g3, g4,
             wo_ref, ws_ref, bo_ref, bs_ref,
             hpl_ref, hmi_ref, mx_ref):
        r = pl.program_id(0)
        dinv = _dinv_from_partials(p_ref[...])
        q1 = (a1[0] + a1[1] + g1[...]) * dinv
        q2 = (a2[0] + a2[1] + g2[...]) * dinv
        q3 = (a3[0] + a3[1] + g3[...]) * dinv
        q4 = (a4[0] + a4[1] + g4[...]) * dinv
        wo = wo_ref[...]
        ws = ws_ref[...]
        bo = bo_ref[...]
        bs = bs_ref[...]
        dot = lambda a, w: jnp.dot(a, w, preferred_element_type=jnp.float32)
        hq_pl = jnp.concatenate([q1, q2], axis=1)
        hq_mi = jnp.concatenate([q3, q4], axis=1)
        h_pl = jnp.concatenate([dot(hq_pl, wo) + bo, dot(hq_pl, ws) + bs], axis=1)
        h_mi = jnp.concatenate([dot(hq_mi, wo) + bo, dot(hq_mi, ws) + bs], axis=1)
        hpl_ref[...] = h_pl
        hmi_ref[...] = h_mi

        row = r * R + lax.broadcasted_iota(jnp.int32, (R, 1), 0)
        neg = jnp.full(h_pl.shape, -jnp.inf, jnp.float32)
        masked = jnp.where(row < N, h_pl, neg)
        bm = jnp.broadcast_to(jnp.max(masked, axis=0)[None, :], (8, 2 * D))

        @pl.when(r == 0)
        def _():
            mx_ref[...] = bm

        @pl.when(r > 0)
        def _():
            mx_ref[...] = jnp.maximum(mx_ref[...], bm)

    grid = NP // R
    blk = lambda r: (r, 0)
    acc_spec = pl.BlockSpec((2, R, D), lambda r: (0, r, 0))
    g_spec = pl.BlockSpec((R, D), blk)
    w_spec = pl.BlockSpec((2 * D, D), lambda r: (0, 0))
    b_spec = pl.BlockSpec((1, D), lambda r: (0, 0))
    return pl.pallas_call(
        body,
        grid=(grid,),
        in_specs=[pl.BlockSpec((32, R), lambda r: (0, r))]
        + [acc_spec] * 4 + [g_spec] * 4 + [w_spec, w_spec, b_spec, b_spec],
        out_specs=[
            pl.BlockSpec((R, 2 * D), blk),
            pl.BlockSpec((R, 2 * D), blk),
            pl.BlockSpec((8, 2 * D), lambda r: (0, 0)),
        ],
        out_shape=[
            jax.ShapeDtypeStruct((NP, 2 * D), jnp.float32),
            jax.ShapeDtypeStruct((NP, 2 * D), jnp.float32),
            jax.ShapeDtypeStruct((8, 2 * D), jnp.float32),
        ],
    )(partials, *accs, *gs, W_o2, W_s2, b_o2, b_s2)


def _tc_disc(mx, W_disc, b_disc, h_pl, h_mi, NP):
    def body(mx_ref, w_ref, b_ref, hpl_ref, hmi_ref, out_ref):
        c = jax.nn.sigmoid(jnp.max(mx_ref[...], axis=0))
        wc = jnp.dot(w_ref[...], c[:, None], preferred_element_type=jnp.float32)
        b = b_ref[0, 0]
        sc1 = jnp.dot(hpl_ref[...], wc, preferred_element_type=jnp.float32)
        sc2 = jnp.dot(hmi_ref[...], wc, preferred_element_type=jnp.float32)
        out_ref[...] = jnp.concatenate([sc1, sc2], axis=1) + b

    grid = NP // R
    return pl.pallas_call(
        body,
        grid=(grid,),
        in_specs=[
            pl.BlockSpec((8, 2 * D), lambda r: (0, 0)),
            pl.BlockSpec((2 * D, 2 * D), lambda r: (0, 0)),
            pl.BlockSpec((1, 1), lambda r: (0, 0), memory_space=pltpu.SMEM),
            pl.BlockSpec((R, 2 * D), lambda r: (r, 0)),
            pl.BlockSpec((R, 2 * D), lambda r: (r, 0)),
        ],
        out_specs=pl.BlockSpec((R, 2), lambda r: (r, 0)),
        out_shape=jax.ShapeDtypeStruct((NP, 2), jnp.float32),
    )(mx, W_disc, b_disc, h_pl, h_mi)


def kernel(x, x_shuf, edge_index, W_o1, b_o1, W_o2, b_o2,
           W_s1, b_s1, W_s2, b_s2, W_disc, b_disc):
    N, _ = x.shape
    E = edge_index.shape[1]
    NP = ((N + R - 1) // R) * R
    EP = ((E + 65535) // 65536) * 65536

    src = edge_index[0]
    dst = edge_index[1]
    pad_e = EP - E
    src_p = jnp.concatenate([src, jnp.zeros((pad_e,), jnp.int32)])
    dst_p = jnp.concatenate([dst, jnp.full((pad_e,), NP - 1, jnp.int32)])
    xp = jnp.pad(x, ((0, NP - N), (0, 0)))
    xsp = jnp.pad(x_shuf, ((0, NP - N), (0, 0)))
    zeros_blk = jnp.zeros((ZR, D), jnp.float32)
    b_o1r = b_o1.reshape(1, D)
    b_s1r = b_s1.reshape(1, D)
    b_o2r = b_o2.reshape(1, D)
    b_s2r = b_s2.reshape(1, D)
    b_discr = b_disc.reshape(1, 1)

    partials = _make_deg_kernel(EP, NP)(dst_p)

    g1x, g1s = _tc_scale(partials, xp, xsp, NP)

    acc1x, acc1s = _make_prop_kernel(2, EP, NP)(src_p, dst_p, zeros_blk, g1x, g1s)

    g2 = _tc_layer1(partials, acc1x, acc1s, g1x, g1s, W_o1, W_s1, b_o1r, b_s1r, NP)

    acc2 = _make_prop_kernel(4, EP, NP)(src_p, dst_p, zeros_blk, *g2)

    h_pl, h_mi, mx = _tc_layer2(partials, acc2, g2, W_o2, W_s2, b_o2r, b_s2r, NP, N)

    logits = _tc_disc(mx, W_disc, b_discr, h_pl, h_mi, NP)
    return logits[:N]


# speedup vs baseline: 6.4097x; 1.0434x over previous
import functools

import jax
import jax.numpy as jnp
from jax import lax
from jax.experimental import pallas as pl
from jax.experimental.pallas import tpu as pltpu
from jax.experimental.pallas import tpu_sc as plsc

D = 128
R = 256
ZR = 64
CH = 128
DEG_CH = 2048


def _mesh():
    return plsc.VectorSubcoreMesh(core_axis_name="c", subcore_axis_name="s")


def _make_deg_kernel(EP, NP):
    per_w = EP // 32

    @functools.partial(
        pl.kernel,
        mesh=_mesh(),
        out_type=jax.ShapeDtypeStruct((32, NP), jnp.float32),
        scratch_types=[
            pltpu.VMEM((NP,), jnp.float32),
            pltpu.VMEM((DEG_CH,), jnp.int32),
        ],
        compiler_params=pltpu.CompilerParams(needs_layout_passes=False),
    )
    def deg_kernel(dst_hbm, out_hbm, acc_v, ids_v):
        cid = lax.axis_index("c")
        sid = lax.axis_index("s")
        wid = sid * 2 + cid
        base = wid * per_w

        def zero_body(i, carry):
            acc_v[pl.ds(i * 16, 16)] = jnp.zeros((16,), jnp.float32)
            return carry

        lax.fori_loop(0, NP // 16, zero_body, 0)

        ones = jnp.ones((16,), jnp.float32)

        def chunk_body(c, carry):
            pltpu.sync_copy(dst_hbm.at[pl.ds(base + c * DEG_CH, DEG_CH)], ids_v)

            def scat_body(i, carry2):
                idx = ids_v[pl.ds(i * 16, 16)]
                plsc.addupdate_scatter(acc_v, [idx], ones)
                return carry2

            lax.fori_loop(0, DEG_CH // 16, scat_body, 0)
            return carry

        lax.fori_loop(0, per_w // DEG_CH, chunk_body, 0)
        pltpu.sync_copy(acc_v, out_hbm.at[wid])

    return deg_kernel


def _make_prop_kernel(T, EP, NP):
    per_w = EP // 32
    n_chunks = per_w // CH
    rows_w = NP // 16

    out_types = [jax.ShapeDtypeStruct((2, NP, D), jnp.float32) for _ in range(T)]
    scratch_types = [
        pltpu.VMEM((CH,), jnp.int32),
        pltpu.VMEM((CH,), jnp.int32),
        pltpu.VMEM((CH,), jnp.int32),
        pltpu.VMEM((CH,), jnp.int32),
        pltpu.VMEM((CH, D), jnp.float32),
        pltpu.VMEM((CH, D), jnp.float32),
        pltpu.VMEM((ZR, D), jnp.float32),
        pltpu.VMEM_SHARED((NP, D), jnp.float32),
        pltpu.SemaphoreType.DMA,
        pltpu.SemaphoreType.DMA,
    ]

    @functools.partial(
        pl.kernel,
        mesh=_mesh(),
        out_type=out_types,
        scratch_types=scratch_types,
    )
    def prop_kernel(src_hbm, dst_hbm, zeros_hbm, *rest):
        tables = rest[:T]
        outs = rest[T:2 * T]
        (sidx0, sidx1, didx0, didx1, rows0, rows1,
         zero_v, acc_sh, sem0, sem1) = rest[2 * T:]
        sidx = (sidx0, sidx1)
        didx = (didx0, didx1)
        rows = (rows0, rows1)
        sems = (sem0, sem1)

        cid = lax.axis_index("c")
        sid = lax.axis_index("s")
        wid = sid * 2 + cid
        ebase = wid * per_w
        row0 = sid * rows_w

        pltpu.sync_copy(zeros_hbm, zero_v)

        for t in range(T):
            def zero_body(j, carry):
                pltpu.sync_copy(zero_v, acc_sh.at[pl.ds(row0 + j * ZR, ZR)])
                return carry

            lax.fori_loop(0, rows_w // ZR, zero_body, 0)
            plsc.subcore_barrier()

            table = tables[t]

            pltpu.sync_copy(src_hbm.at[pl.ds(ebase, CH)], sidx[0])
            pltpu.sync_copy(dst_hbm.at[pl.ds(ebase, CH)], didx[0])
            pltpu.async_copy(table.at[sidx[0]], rows[0], sems[0])

            def pair_body(i, carry):
                for b in (0, 1):
                    c = 2 * i + b
                    ob = 1 - b
                    off = ebase + (c + 1) * CH
                    pltpu.sync_copy(src_hbm.at[pl.ds(off, CH)], sidx[ob])
                    pltpu.sync_copy(dst_hbm.at[pl.ds(off, CH)], didx[ob])
                    pltpu.make_async_copy(table.at[sidx[b]], rows[b], sems[b]).wait()
                    pltpu.async_copy(table.at[sidx[ob]], rows[ob], sems[ob])
                    pltpu.sync_copy(rows[b], acc_sh.at[didx[b]], add=True)
                return carry

            lax.fori_loop(0, n_chunks // 2, pair_body, 0)
            pltpu.make_async_copy(table.at[sidx[0]], rows[0], sems[0]).wait()
            plsc.subcore_barrier()

            def flush_body(j, carry):
                r = row0 + j * CH
                pltpu.sync_copy(acc_sh.at[pl.ds(r, CH)], rows0)
                pltpu.sync_copy(rows0, outs[t].at[cid, pl.ds(r, CH)])
                return carry

            lax.fori_loop(0, rows_w // CH, flush_body, 0)

    return prop_kernel


def _dinv_from_partials(p_blk):
    deg = jnp.sum(p_blk, axis=0) + 1.0
    return lax.rsqrt(deg)[:, None]


def _tc_scale(partials, xp, xsp, NP):
    def body(p_ref, x_ref, xs_ref, gx_ref, gs_ref):
        dinv = _dinv_from_partials(p_ref[...])
        gx_ref[...] = x_ref[...] * dinv
        gs_ref[...] = xs_ref[...] * dinv

    grid = NP // R
    blk = lambda r: (r, 0)
    return pl.pallas_call(
        body,
        grid=(grid,),
        in_specs=[
            pl.BlockSpec((32, R), lambda r: (0, r)),
            pl.BlockSpec((R, D), blk),
            pl.BlockSpec((R, D), blk),
        ],
        out_specs=[pl.BlockSpec((R, D), blk)] * 2,
        out_shape=[jax.ShapeDtypeStruct((NP, D), jnp.float32)] * 2,
    )(partials, xp, xsp)


def _tc_layer1(partials, acc1x, acc1s, g1x, g1s, W_o1, W_s1, b_o1, b_s1, NP):
    def body(p_ref, ax_ref, as_ref, gx_ref, gs_ref,
             wo_ref, ws_ref, bo_ref, bs_ref,
             o1_ref, o2_ref, o3_ref, o4_ref):
        dinv = _dinv_from_partials(p_ref[...])
        p1x = (ax_ref[0] + ax_ref[1] + gx_ref[...]) * dinv
        p1s = (as_ref[0] + as_ref[1] + gs_ref[...]) * dinv
        wo = wo_ref[...]
        ws = ws_ref[...]
        bo = bo_ref[...]
        bs = bs_ref[...]
        dot = lambda a, w: jnp.dot(a, w, preferred_element_type=jnp.float32)
        o1_ref[...] = jax.nn.relu(dot(p1x, wo) + bo) * dinv
        o2_ref[...] = jax.nn.relu(dot(p1x, ws) + bs) * dinv
        o3_ref[...] = jax.nn.relu(dot(p1s, wo) + bo) * dinv
        o4_ref[...] = jax.nn.relu(dot(p1s, ws) + bs) * dinv

    grid = NP // R
    blk = lambda r: (r, 0)
    acc_spec = pl.BlockSpec((2, R, D), lambda r: (0, r, 0))
    w_spec = pl.BlockSpec((D, D), lambda r: (0, 0))
    b_spec = pl.BlockSpec((1, D), lambda r: (0, 0))
    return pl.pallas_call(
        body,
        grid=(grid,),
        in_specs=[
            pl.BlockSpec((32, R), lambda r: (0, r)),
            acc_spec, acc_spec,
            pl.BlockSpec((R, D), blk), pl.BlockSpec((R, D), blk),
            w_spec, w_spec, b_spec, b_spec,
        ],
        out_specs=[pl.BlockSpec((R, D), blk)] * 4,
        out_shape=[jax.ShapeDtypeStruct((NP, D), jnp.float32)] * 4,
    )(partials, acc1x, acc1s, g1x, g1s, W_o1, W_s1, b_o1, b_s1)


def _tc_layer2(partials, accs, gs, W_o2, W_s2, b_o2, b_s2, NP, N):
    def body(p_ref, a1, a2, a3, a4, g1, g2, g3, g4,
             wo_ref, ws_ref, bo_ref, bs_ref,
             hpl_ref, hmi_ref, mx_ref):
        r = pl.program_id(0)
        dinv = _dinv_from_partials(p_ref[...])
        q1 = (a1[0] + a1[1] + g1[...]) * dinv
        q2 = (a2[0] + a2[1] + g2[...]) * dinv
        q3 = (a3[0] + a3[1] + g3[...]) * dinv
        q4 = (a4[0] + a4[1] + g4[...]) * dinv
        wo = wo_ref[...]
        ws = ws_ref[...]
        bo = bo_ref[...]
        bs = bs_ref[...]
        dot = lambda a, w: jnp.dot(a, w, preferred_element_type=jnp.float32)
        hq_pl = jnp.concatenate([q1, q2], axis=1)
        hq_mi = jnp.concatenate([q3, q4], axis=1)
        h_pl = jnp.concatenate([dot(hq_pl, wo) + bo, dot(hq_pl, ws) + bs], axis=1)
        h_mi = jnp.concatenate([dot(hq_mi, wo) + bo, dot(hq_mi, ws) + bs], axis=1)
        hpl_ref[...] = h_pl
        hmi_ref[...] = h_mi

        row = r * R + lax.broadcasted_iota(jnp.int32, (R, 1), 0)
        neg = jnp.full(h_pl.shape, -jnp.inf, jnp.float32)
        masked = jnp.where(row < N, h_pl, neg)
        bm = jnp.broadcast_to(jnp.max(masked, axis=0)[None, :], (8, 2 * D))

        @pl.when(r == 0)
        def _():
            mx_ref[...] = bm

        @pl.when(r > 0)
        def _():
            mx_ref[...] = jnp.maximum(mx_ref[...], bm)

    grid = NP // R
    blk = lambda r: (r, 0)
    acc_spec = pl.BlockSpec((2, R, D), lambda r: (0, r, 0))
    g_spec = pl.BlockSpec((R, D), blk)
    w_spec = pl.BlockSpec((2 * D, D), lambda r: (0, 0))
    b_spec = pl.BlockSpec((1, D), lambda r: (0, 0))
    return pl.pallas_call(
        body,
        grid=(grid,),
        in_specs=[pl.BlockSpec((32, R), lambda r: (0, r))]
        + [acc_spec] * 4 + [g_spec] * 4 + [w_spec, w_spec, b_spec, b_spec],
        out_specs=[
            pl.BlockSpec((R, 2 * D), blk),
            pl.BlockSpec((R, 2 * D), blk),
            pl.BlockSpec((8, 2 * D), lambda r: (0, 0)),
        ],
        out_shape=[
            jax.ShapeDtypeStruct((NP, 2 * D), jnp.float32),
            jax.ShapeDtypeStruct((NP, 2 * D), jnp.float32),
            jax.ShapeDtypeStruct((8, 2 * D), jnp.float32),
        ],
    )(partials, *accs, *gs, W_o2, W_s2, b_o2, b_s2)


def _tc_disc(mx, W_disc, b_disc, h_pl, h_mi, NP):
    def body(mx_ref, w_ref, b_ref, hpl_ref, hmi_ref, out_ref):
        c = jax.nn.sigmoid(jnp.max(mx_ref[...], axis=0))
        wc = jnp.dot(w_ref[...], c[:, None], preferred_element_type=jnp.float32)
        b = b_ref[0, 0]
        sc1 = jnp.dot(hpl_ref[...], wc, preferred_element_type=jnp.float32)
        sc2 = jnp.dot(hmi_ref[...], wc, preferred_element_type=jnp.float32)
        out_ref[...] = jnp.concatenate([sc1, sc2], axis=1) + b

    grid = NP // R
    return pl.pallas_call(
        body,
        grid=(grid,),
        in_specs=[
            pl.BlockSpec((8, 2 * D), lambda r: (0, 0)),
            pl.BlockSpec((2 * D, 2 * D), lambda r: (0, 0)),
            pl.BlockSpec((1, 1), lambda r: (0, 0), memory_space=pltpu.SMEM),
            pl.BlockSpec((R, 2 * D), lambda r: (r, 0)),
            pl.BlockSpec((R, 2 * D), lambda r: (r, 0)),
        ],
        out_specs=pl.BlockSpec((R, 2), lambda r: (r, 0)),
        out_shape=jax.ShapeDtypeStruct((NP, 2), jnp.float32),
    )(mx, W_disc, b_disc, h_pl, h_mi)


def kernel(x, x_shuf, edge_index, W_o1, b_o1, W_o2, b_o2,
           W_s1, b_s1, W_s2, b_s2, W_disc, b_disc):
    N, _ = x.shape
    E = edge_index.shape[1]
    NP = ((N + R - 1) // R) * R
    EP = ((E + 65535) // 65536) * 65536

    src = edge_index[0]
    dst = edge_index[1]
    pad_e = EP + CH - E
    src_p = jnp.concatenate([src, jnp.zeros((pad_e,), jnp.int32)])
    dst_p = jnp.concatenate([dst, jnp.full((pad_e,), NP - 1, jnp.int32)])
    xp = jnp.pad(x, ((0, NP - N), (0, 0)))
    xsp = jnp.pad(x_shuf, ((0, NP - N), (0, 0)))
    zeros_blk = jnp.zeros((ZR, D), jnp.float32)
    b_o1r = b_o1.reshape(1, D)
    b_s1r = b_s1.reshape(1, D)
    b_o2r = b_o2.reshape(1, D)
    b_s2r = b_s2.reshape(1, D)
    b_discr = b_disc.reshape(1, 1)

    partials = _make_deg_kernel(EP, NP)(dst_p)

    g1x, g1s = _tc_scale(partials, xp, xsp, NP)

    acc1x, acc1s = _make_prop_kernel(2, EP, NP)(src_p, dst_p, zeros_blk, g1x, g1s)

    g2 = _tc_layer1(partials, acc1x, acc1s, g1x, g1s, W_o1, W_s1, b_o1r, b_s1r, NP)

    acc2 = _make_prop_kernel(4, EP, NP)(src_p, dst_p, zeros_blk, *g2)

    h_pl, h_mi, mx = _tc_layer2(partials, acc2, g2, W_o2, W_s2, b_o2r, b_s2r, NP, N)

    logits = _tc_disc(mx, W_disc, b_discr, h_pl, h_mi, NP)
    return logits[:N]


# speedup vs baseline: 7.2503x; 1.1312x over previous
import functools

import jax
import jax.numpy as jnp
from jax import lax
from jax.experimental import pallas as pl
from jax.experimental.pallas import tpu as pltpu
from jax.experimental.pallas import tpu_sc as plsc

D = 128
R = 256
ZR = 8
CH = 64
DEG_CH = 2048


def _mesh():
    return plsc.VectorSubcoreMesh(core_axis_name="c", subcore_axis_name="s")


def _make_deg_kernel(EP, NP):
    per_w = EP // 32

    @functools.partial(
        pl.kernel,
        mesh=_mesh(),
        out_type=jax.ShapeDtypeStruct((32, NP), jnp.float32),
        scratch_types=[
            pltpu.VMEM((NP,), jnp.float32),
            pltpu.VMEM((DEG_CH,), jnp.int32),
        ],
        compiler_params=pltpu.CompilerParams(needs_layout_passes=False),
    )
    def deg_kernel(dst_hbm, out_hbm, acc_v, ids_v):
        cid = lax.axis_index("c")
        sid = lax.axis_index("s")
        wid = sid * 2 + cid
        base = wid * per_w

        def zero_body(i, carry):
            acc_v[pl.ds(i * 16, 16)] = jnp.zeros((16,), jnp.float32)
            return carry

        lax.fori_loop(0, NP // 16, zero_body, 0)

        ones = jnp.ones((16,), jnp.float32)

        def chunk_body(c, carry):
            pltpu.sync_copy(dst_hbm.at[pl.ds(base + c * DEG_CH, DEG_CH)], ids_v)

            def scat_body(i, carry2):
                idx = ids_v[pl.ds(i * 16, 16)]
                plsc.addupdate_scatter(acc_v, [idx], ones)
                return carry2

            lax.fori_loop(0, DEG_CH // 16, scat_body, 0)
            return carry

        lax.fori_loop(0, per_w // DEG_CH, chunk_body, 0)
        pltpu.sync_copy(acc_v, out_hbm.at[wid])

    return deg_kernel


def _make_prop_kernel(T, EP, NP):
    per_w = EP // 32
    n_chunks = per_w // CH
    rows_w = NP // 16

    BK = 16
    n_blocks = n_chunks // BK
    NBUF = 4

    out_types = [jax.ShapeDtypeStruct((2, NP, D), jnp.float32) for _ in range(T)]
    scratch_types = (
        [pltpu.VMEM((BK, CH), jnp.int32),
         pltpu.VMEM((BK, CH), jnp.int32)]
        + [pltpu.VMEM((CH, D), jnp.float32) for _ in range(NBUF)]
        + [pltpu.VMEM((ZR, D), jnp.float32),
           pltpu.VMEM_SHARED((NP, D), jnp.float32)]
        + [pltpu.SemaphoreType.DMA for _ in range(2 * NBUF)]
    )

    @functools.partial(
        pl.kernel,
        mesh=_mesh(),
        out_type=out_types,
        scratch_types=scratch_types,
    )
    def prop_kernel(src_hbm, dst_hbm, zeros_hbm, *rest):
        tables = rest[:T]
        outs = rest[T:2 * T]
        rest = rest[2 * T:]
        sidx_v, didx_v = rest[0], rest[1]
        rows = rest[2:2 + NBUF]
        zero_v, acc_sh = rest[2 + NBUF], rest[3 + NBUF]
        sem_g = rest[4 + NBUF:4 + 2 * NBUF]
        sem_s = rest[4 + 2 * NBUF:4 + 3 * NBUF]

        cid = lax.axis_index("c")
        sid = lax.axis_index("s")
        wid = sid * 2 + cid
        cbase = wid * n_chunks
        row0 = sid * rows_w

        pltpu.sync_copy(zeros_hbm, zero_v)

        for t in range(T):
            def zero_body(j, carry):
                pltpu.sync_copy(zero_v, acc_sh.at[pl.ds(row0 + j * ZR, ZR)])
                return carry

            lax.fori_loop(0, rows_w // ZR, zero_body, 0)
            plsc.subcore_barrier()

            table = tables[t]

            def gath(k):
                return pltpu.make_async_copy(
                    table.at[sidx_v.at[k]], rows[k % NBUF], sem_g[k % NBUF])

            def scat(k):
                return pltpu.make_async_copy(
                    rows[k % NBUF], acc_sh.at[didx_v.at[k]], sem_s[k % NBUF])

            def block_body(i, carry):
                crow = cbase + i * BK
                pltpu.sync_copy(src_hbm.at[pl.ds(crow, BK)], sidx_v)
                pltpu.sync_copy(dst_hbm.at[pl.ds(crow, BK)], didx_v)
                gath(0).start()
                gath(1).start()
                for k in range(BK):
                    gath(k).wait()
                    if k + 2 < BK:
                        if k >= 2:
                            scat(k - 2).wait()
                        gath(k + 2).start()
                    scat(k).start(add=True)
                for k in range(BK - 4, BK):
                    scat(k).wait()
                return carry

            lax.fori_loop(0, n_blocks, block_body, 0)
            plsc.subcore_barrier()

            def flush_body(j, carry):
                r = row0 + j * CH
                pltpu.sync_copy(acc_sh.at[pl.ds(r, CH)], rows[0])
                pltpu.sync_copy(rows[0], outs[t].at[cid, pl.ds(r, CH)])
                return carry

            lax.fori_loop(0, rows_w // CH, flush_body, 0)

    return prop_kernel


def _dinv_from_partials(p_blk):
    deg = jnp.sum(p_blk, axis=0) + 1.0
    return lax.rsqrt(deg)[:, None]


def _tc_scale(partials, xp, xsp, NP):
    def body(p_ref, x_ref, xs_ref, gx_ref, gs_ref):
        dinv = _dinv_from_partials(p_ref[...])
        gx_ref[...] = x_ref[...] * dinv
        gs_ref[...] = xs_ref[...] * dinv

    grid = NP // R
    blk = lambda r: (r, 0)
    return pl.pallas_call(
        body,
        grid=(grid,),
        in_specs=[
            pl.BlockSpec((32, R), lambda r: (0, r)),
            pl.BlockSpec((R, D), blk),
            pl.BlockSpec((R, D), blk),
        ],
        out_specs=[pl.BlockSpec((R, D), blk)] * 2,
        out_shape=[jax.ShapeDtypeStruct((NP, D), jnp.float32)] * 2,
    )(partials, xp, xsp)


def _tc_layer1(partials, acc1x, acc1s, g1x, g1s, W_o1, W_s1, b_o1, b_s1, NP):
    def body(p_ref, ax_ref, as_ref, gx_ref, gs_ref,
             wo_ref, ws_ref, bo_ref, bs_ref,
             o1_ref, o2_ref, o3_ref, o4_ref):
        dinv = _dinv_from_partials(p_ref[...])
        p1x = (ax_ref[0] + ax_ref[1] + gx_ref[...]) * dinv
        p1s = (as_ref[0] + as_ref[1] + gs_ref[...]) * dinv
        wo = wo_ref[...]
        ws = ws_ref[...]
        bo = bo_ref[...]
        bs = bs_ref[...]
        dot = lambda a, w: jnp.dot(a, w, preferred_element_type=jnp.float32)
        o1_ref[...] = jax.nn.relu(dot(p1x, wo) + bo) * dinv
        o2_ref[...] = jax.nn.relu(dot(p1x, ws) + bs) * dinv
        o3_ref[...] = jax.nn.relu(dot(p1s, wo) + bo) * dinv
        o4_ref[...] = jax.nn.relu(dot(p1s, ws) + bs) * dinv

    grid = NP // R
    blk = lambda r: (r, 0)
    acc_spec = pl.BlockSpec((2, R, D), lambda r: (0, r, 0))
    w_spec = pl.BlockSpec((D, D), lambda r: (0, 0))
    b_spec = pl.BlockSpec((1, D), lambda r: (0, 0))
    return pl.pallas_call(
        body,
        grid=(grid,),
        in_specs=[
            pl.BlockSpec((32, R), lambda r: (0, r)),
            acc_spec, acc_spec,
            pl.BlockSpec((R, D), blk), pl.BlockSpec((R, D), blk),
            w_spec, w_spec, b_spec, b_spec,
        ],
        out_specs=[pl.BlockSpec((R, D), blk)] * 4,
        out_shape=[jax.ShapeDtypeStruct((NP, D), jnp.float32)] * 4,
    )(partials, acc1x, acc1s, g1x, g1s, W_o1, W_s1, b_o1, b_s1)


def _tc_layer2(partials, accs, gs, W_o2, W_s2, b_o2, b_s2, NP, N):
    def body(p_ref, a1, a2, a3, a4, g1, g2, g3, g4,
             wo_ref, ws_ref, bo_ref, bs_ref,
             hpl_ref, hmi_ref, mx_ref):
        r = pl.program_id(0)
        dinv = _dinv_from_partials(p_ref[...])
        q1 = (a1[0] + a1[1] + g1[...]) * dinv
        q2 = (a2[0] + a2[1] + g2[...]) * dinv
        q3 = (a3[0] + a3[1] + g3[...]) * dinv
        q4 = (a4[0] + a4[1] + g4[...]) * dinv
        wo = wo_ref[...]
        ws = ws_ref[...]
        bo = bo_ref[...]
        bs = bs_ref[...]
        dot = lambda a, w: jnp.dot(a, w, preferred_element_type=jnp.float32)
        hq_pl = jnp.concatenate([q1, q2], axis=1)
        hq_mi = jnp.concatenate([q3, q4], axis=1)
        h_pl = jnp.concatenate([dot(hq_pl, wo) + bo, dot(hq_pl, ws) + bs], axis=1)
        h_mi = jnp.concatenate([dot(hq_mi, wo) + bo, dot(hq_mi, ws) + bs], axis=1)
        hpl_ref[...] = h_pl
        hmi_ref[...] = h_mi

        row = r * R + lax.broadcasted_iota(jnp.int32, (R, 1), 0)
        neg = jnp.full(h_pl.shape, -jnp.inf, jnp.float32)
        masked = jnp.where(row < N, h_pl, neg)
        bm = jnp.broadcast_to(jnp.max(masked, axis=0)[None, :], (8, 2 * D))

        @pl.when(r == 0)
        def _():
            mx_ref[...] = bm

        @pl.when(r > 0)
        def _():
            mx_ref[...] = jnp.maximum(mx_ref[...], bm)

    grid = NP // R
    blk = lambda r: (r, 0)
    acc_spec = pl.BlockSpec((2, R, D), lambda r: (0, r, 0))
    g_spec = pl.BlockSpec((R, D), blk)
    w_spec = pl.BlockSpec((2 * D, D), lambda r: (0, 0))
    b_spec = pl.BlockSpec((1, D), lambda r: (0, 0))
    return pl.pallas_call(
        body,
        grid=(grid,),
        in_specs=[pl.BlockSpec((32, R), lambda r: (0, r))]
        + [acc_spec] * 4 + [g_spec] * 4 + [w_spec, w_spec, b_spec, b_spec],
        out_specs=[
            pl.BlockSpec((R, 2 * D), blk),
            pl.BlockSpec((R, 2 * D), blk),
            pl.BlockSpec((8, 2 * D), lambda r: (0, 0)),
        ],
        out_shape=[
            jax.ShapeDtypeStruct((NP, 2 * D), jnp.float32),
            jax.ShapeDtypeStruct((NP, 2 * D), jnp.float32),
            jax.ShapeDtypeStruct((8, 2 * D), jnp.float32),
        ],
    )(partials, *accs, *gs, W_o2, W_s2, b_o2, b_s2)


def _tc_disc(mx, W_disc, b_disc, h_pl, h_mi, NP):
    def body(mx_ref, w_ref, b_ref, hpl_ref, hmi_ref, out_ref):
        c = jax.nn.sigmoid(jnp.max(mx_ref[...], axis=0))
        wc = jnp.dot(w_ref[...], c[:, None], preferred_element_type=jnp.float32)
        b = b_ref[0, 0]
        sc1 = jnp.dot(hpl_ref[...], wc, preferred_element_type=jnp.float32)
        sc2 = jnp.dot(hmi_ref[...], wc, preferred_element_type=jnp.float32)
        out_ref[...] = jnp.concatenate([sc1, sc2], axis=1) + b

    grid = NP // R
    return pl.pallas_call(
        body,
        grid=(grid,),
        in_specs=[
            pl.BlockSpec((8, 2 * D), lambda r: (0, 0)),
            pl.BlockSpec((2 * D, 2 * D), lambda r: (0, 0)),
            pl.BlockSpec((1, 1), lambda r: (0, 0), memory_space=pltpu.SMEM),
            pl.BlockSpec((R, 2 * D), lambda r: (r, 0)),
            pl.BlockSpec((R, 2 * D), lambda r: (r, 0)),
        ],
        out_specs=pl.BlockSpec((R, 2), lambda r: (r, 0)),
        out_shape=jax.ShapeDtypeStruct((NP, 2), jnp.float32),
    )(mx, W_disc, b_disc, h_pl, h_mi)


def kernel(x, x_shuf, edge_index, W_o1, b_o1, W_o2, b_o2,
           W_s1, b_s1, W_s2, b_s2, W_disc, b_disc):
    N, _ = x.shape
    E = edge_index.shape[1]
    NP = ((N + R - 1) // R) * R
    EP = ((E + 65535) // 65536) * 65536

    src = edge_index[0]
    dst = edge_index[1]
    pad_e = EP - E
    src_p = jnp.concatenate([src, jnp.zeros((pad_e,), jnp.int32)])
    dst_p = jnp.concatenate([dst, jnp.full((pad_e,), NP - 1, jnp.int32)])
    src2 = src_p.reshape(EP // CH, CH)
    dst2 = dst_p.reshape(EP // CH, CH)
    xp = jnp.pad(x, ((0, NP - N), (0, 0)))
    xsp = jnp.pad(x_shuf, ((0, NP - N), (0, 0)))
    zeros_blk = jnp.zeros((ZR, D), jnp.float32)
    b_o1r = b_o1.reshape(1, D)
    b_s1r = b_s1.reshape(1, D)
    b_o2r = b_o2.reshape(1, D)
    b_s2r = b_s2.reshape(1, D)
    b_discr = b_disc.reshape(1, 1)

    partials = _make_deg_kernel(EP, NP)(dst_p)

    g1x, g1s = _tc_scale(partials, xp, xsp, NP)

    acc1x, acc1s = _make_prop_kernel(2, EP, NP)(src2, dst2, zeros_blk, g1x, g1s)

    g2 = _tc_layer1(partials, acc1x, acc1s, g1x, g1s, W_o1, W_s1, b_o1r, b_s1r, NP)

    acc2 = _make_prop_kernel(4, EP, NP)(src2, dst2, zeros_blk, *g2)

    h_pl, h_mi, mx = _tc_layer2(partials, acc2, g2, W_o2, W_s2, b_o2r, b_s2r, NP, N)

    logits = _tc_disc(mx, W_disc, b_discr, h_pl, h_mi, NP)
    return logits[:N]
